# baseline (speedup 1.0000x reference)
import functools

import jax
import jax.numpy as jnp
from jax import lax
from jax.experimental import pallas as pl
from jax.experimental.pallas import tpu as pltpu
from jax.experimental.pallas import tpu_sc as plsc

B, S, D = 4, 4096, 1024
NC, NS, L = 2, 16, 16
NW = NC * NS
SPW = S // NW
T = 16
NCH = SPW // T
NSTEP = NCH * B
NVEC = T * D // L
NXB = 5
LOOKAHEAD = 4

_mesh = plsc.VectorSubcoreMesh(
    core_axis_name="c", subcore_axis_name="s", num_cores=NC, num_subcores=NS
)


@functools.partial(
    pl.kernel,
    out_type=jax.ShapeDtypeStruct((B, S, D), jnp.float32),
    mesh=_mesh,
    compiler_params=pltpu.CompilerParams(use_tc_tiling_on_sc=True),
    scratch_types=[
        [pltpu.VMEM((T, D), jnp.float32)] * 2,
        [pltpu.VMEM((T, D), jnp.float32)] * NXB,
        [pltpu.SemaphoreType.DMA] * 2,
        [pltpu.SemaphoreType.DMA] * NXB,
        [pltpu.SemaphoreType.DMA] * NXB,
    ],
)
def _sc_add(x_hbm, pos_hbm, out_hbm, p_v, x_v, sem_p, sem_x, sem_o):
    wid = lax.axis_index("s") * NC + lax.axis_index("c")
    s0 = wid * SPW

    def pos_load(ci):
        return pltpu.make_async_copy(
            pos_hbm.at[pl.ds(s0 + ci * T, T)], p_v[ci % 2], sem_p[ci % 2]
        )

    def x_load(step):
        ci, b = step // B, step % B
        return pltpu.make_async_copy(
            x_hbm.at[b, pl.ds(s0 + ci * T, T)], x_v[step % NXB], sem_x[step % NXB]
        )

    def out_store(step):
        ci, b = step // B, step % B
        return pltpu.make_async_copy(
            x_v[step % NXB], out_hbm.at[b, pl.ds(s0 + ci * T, T)], sem_o[step % NXB]
        )

    pos_load(0).start()
    for s in range(LOOKAHEAD):
        x_load(s).start()

    for step in range(NSTEP):
        ci = step // B
        if step % B == 0:
            pos_load(ci).wait()
            if ci + 1 < NCH:
                pos_load(ci + 1).start()
        x_load(step).wait()

        pv = p_v[ci % 2]
        xv = x_v[step % NXB]

        @plsc.parallel_loop(0, NVEC, unroll=16)
        def _acc(i):
            r = i // (D // L)
            c = (i % (D // L)) * L
            sl = pl.ds(c, L)
            plsc.addupdate(xv.at[r, sl], pv[r, sl])

        out_store(step).start()
        if step + LOOKAHEAD < NSTEP:
            prev = step + LOOKAHEAD - NXB
            if prev >= 0:
                out_store(prev).wait()
            x_load(step + LOOKAHEAD).start()

    for step in range(NSTEP - NXB, NSTEP):
        out_store(step).wait()


def kernel(x, pos_table):
    return _sc_add(x, pos_table)


# speedup vs baseline: 1.0215x; 1.0215x over previous
import functools

import jax
import jax.numpy as jnp
from jax import lax
from jax.experimental import pallas as pl
from jax.experimental.pallas import tpu as pltpu
from jax.experimental.pallas import tpu_sc as plsc

B, S, D = 4, 4096, 1024
NC, NS, L = 2, 16, 16
NW = NC * NS
SPW = S // NW
T = 16
NCH = SPW // T
NSTEP = NCH * B
NVEC = T * D // L
NXB = 5
LOOKAHEAD = 4

_mesh = plsc.VectorSubcoreMesh(
    core_axis_name="c", subcore_axis_name="s", num_cores=NC, num_subcores=NS
)


@functools.partial(
    pl.kernel,
    out_type=jax.ShapeDtypeStruct((B, S, D), jnp.float32),
    mesh=_mesh,
    compiler_params=pltpu.CompilerParams(use_tc_tiling_on_sc=True),
    scratch_types=[
        [pltpu.VMEM((T, D), jnp.float32)] * 2,
        [pltpu.VMEM((T, D), jnp.float32)] * NXB,
        [pltpu.SemaphoreType.DMA] * 2,
        [pltpu.SemaphoreType.DMA] * NXB,
        [pltpu.SemaphoreType.DMA] * NXB,
    ],
)
def _sc_add(x_hbm, pos_hbm, out_hbm, p_v, x_v, sem_p, sem_x, sem_o):
    wid = lax.axis_index("s") * NC + lax.axis_index("c")
    s0 = wid * SPW

    def pos_load(ci):
        return pltpu.make_async_copy(
            pos_hbm.at[pl.ds(s0 + ci * T, T)], p_v[ci % 2], sem_p[ci % 2]
        )

    class _Pair:
        def __init__(self, descs):
            self.descs = descs

        def start(self):
            for d in self.descs:
                d.start()

        def wait(self):
            for d in self.descs:
                d.wait()

    H = T // 2

    def x_load(step):
        ci, b = step // B, step % B
        return _Pair([
            pltpu.make_async_copy(
                x_hbm.at[b, pl.ds(s0 + ci * T + h * H, H)],
                x_v[step % NXB].at[pl.ds(h * H, H)],
                sem_x[step % NXB],
            )
            for h in range(2)
        ])

    def out_store(step):
        ci, b = step // B, step % B
        return _Pair([
            pltpu.make_async_copy(
                x_v[step % NXB].at[pl.ds(h * H, H)],
                out_hbm.at[b, pl.ds(s0 + ci * T + h * H, H)],
                sem_o[step % NXB],
            )
            for h in range(2)
        ])

    pos_load(0).start()
    for s in range(LOOKAHEAD):
        x_load(s).start()

    for step in range(NSTEP):
        ci = step // B
        if step % B == 0:
            pos_load(ci).wait()
            if ci + 1 < NCH:
                pos_load(ci + 1).start()
        x_load(step).wait()

        pv = p_v[ci % 2]
        xv = x_v[step % NXB]

        @plsc.parallel_loop(0, NVEC, unroll=8)
        def _acc(i):
            r = i // (D // L)
            c = (i % (D // L)) * L
            sl = pl.ds(c, L)
            plsc.addupdate(xv.at[r, sl], pv[r, sl])

        out_store(step).start()
        if step + LOOKAHEAD < NSTEP:
            prev = step + LOOKAHEAD - NXB
            if prev >= 0:
                out_store(prev).wait()
            x_load(step + LOOKAHEAD).start()

    for step in range(NSTEP - NXB, NSTEP):
        out_store(step).wait()


def kernel(x, pos_table):
    return _sc_add(x, pos_table)


# speedup vs baseline: 1.0487x; 1.0266x over previous
import functools

import jax
import jax.numpy as jnp
from jax import lax
from jax.experimental import pallas as pl
from jax.experimental.pallas import tpu as pltpu
from jax.experimental.pallas import tpu_sc as plsc

B, S, D = 4, 4096, 1024
NC, NS, L = 2, 16, 16
NW = NC * NS
SPW = S // NW
T = 16
NCH = SPW // T
NSTEP = NCH * B
CBYTES = T * D * 4

_mesh = plsc.VectorSubcoreMesh(
    core_axis_name="c", subcore_axis_name="s", num_cores=NC, num_subcores=NS
)


@functools.partial(
    pl.kernel,
    out_type=jax.ShapeDtypeStruct((B, S, D), jnp.float32),
    mesh=_mesh,
    compiler_params=pltpu.CompilerParams(use_tc_tiling_on_sc=True),
    scratch_types=[
        [pltpu.VMEM((T, D), jnp.float32)] * 2,
        [pltpu.VMEM((T, D), jnp.float32)] * 4,
        [pltpu.SemaphoreType.DMA] * 2,
        [pltpu.SemaphoreType.DMA] * 4,
        [pltpu.SemaphoreType.DMA] * 4,
    ],
)
def _sc_add(x_hbm, pos_hbm, out_hbm, p_v, x_v, sem_p, sem_x, sem_o):
    wid = lax.axis_index("s") * NC + lax.axis_index("c")
    s0 = wid * SPW

    def pos_load(ci, par):
        return pltpu.make_async_copy(
            pos_hbm.at[pl.ds(s0 + ci * T, T)], p_v[par], sem_p[par]
        )

    def x_load(ci, b, ring):
        return pltpu.make_async_copy(
            x_hbm.at[b, pl.ds(s0 + ci * T, T)], x_v[ring], sem_x[ring]
        )

    def out_store(ci, b, ring):
        return pltpu.make_async_copy(
            x_v[ring], out_hbm.at[b, pl.ds(s0 + ci * T, T)], sem_o[ring]
        )

    def step_code(g, k, last_group):
        ci = 2 * g + k // 4
        b = k % 4
        ring = k % 4
        par = (k // 4) % 2
        if k % 4 == 0:
            pos_load(ci, par).wait()
            if not (last_group and k >= 4):
                pos_load(ci + 1, 1 - par).start()
        x_load(ci, b, ring).wait()

        pv = p_v[par]
        xv = x_v[ring]

        @plsc.parallel_loop(0, T * D // L, unroll=8)
        def _acc(i):
            r = i // (D // L)
            c = (i % (D // L)) * L
            sl = pl.ds(c, L)
            plsc.addupdate(xv.at[r, sl], pv[r, sl])

        out_store(ci, b, ring).start()
        if not (last_group and k >= 5):
            if k >= 1:
                out_store(2 * g + (k - 1) // 4, (k - 1) % 4, (k - 1) % 4).wait()
            elif last_group:
                out_store(2 * g - 1, 3, 3).wait()
            else:
                @pl.when(g >= 1)
                def _():
                    out_store(2 * g - 1, 3, 3).wait()
            k3 = k + 3
            if k3 < 8:
                ci3 = 2 * g + k3 // 4
            else:
                ci3, k3 = 2 * g + 2, k3 - 8
            x_load(ci3, k3 % 4, (k + 3) % 4).start()

    pos_load(0, 0).start()
    for st in range(3):
        x_load(0, st, st).start()

    def group(g, carry):
        for k in range(8):
            step_code(g, k, last_group=False)
        return carry

    lax.fori_loop(0, 3, group, 0)
    for k in range(8):
        step_code(3, k, last_group=True)

    for k in range(4):
        out_store(7, k, k).wait()


def kernel(x, pos_table):
    return _sc_add(x, pos_table)


# speedup vs baseline: 1.0563x; 1.0073x over previous
import functools

import jax
import jax.numpy as jnp
from jax import lax
from jax.experimental import pallas as pl
from jax.experimental.pallas import tpu as pltpu
from jax.experimental.pallas import tpu_sc as plsc

B, S, D = 4, 4096, 1024
NC, NS, L = 2, 16, 16
NW = NC * NS
SPW = S // NW
T = 16
NCH = SPW // T
NSTEP = NCH * B
NG = NSTEP // 8

_mesh = plsc.VectorSubcoreMesh(
    core_axis_name="c", subcore_axis_name="s", num_cores=NC, num_subcores=NS
)


@functools.partial(
    pl.kernel,
    out_type=jax.ShapeDtypeStruct((B, S, D), jnp.float32),
    mesh=_mesh,
    compiler_params=pltpu.CompilerParams(use_tc_tiling_on_sc=True),
    scratch_types=[
        [pltpu.VMEM((T, D), jnp.float32)] * 2,
        [pltpu.VMEM((T, D), jnp.float32)] * 4,
        [pltpu.SemaphoreType.DMA] * 2,
        [pltpu.SemaphoreType.DMA] * 4,
        [pltpu.SemaphoreType.DMA] * 4,
    ],
)
def _sc_add(x_hbm, pos_hbm, out_hbm, p_v, x_v, sem_p, sem_x, sem_o):
    wid = lax.axis_index("s") * NC + lax.axis_index("c")
    s0 = wid * SPW

    def pos_load(ci, par):
        return pltpu.make_async_copy(
            pos_hbm.at[pl.ds(s0 + ci * T, T)], p_v[par], sem_p[par]
        )

    def x_load(ci, b, ring):
        return pltpu.make_async_copy(
            x_hbm.at[b, pl.ds(s0 + ci * T, T)], x_v[ring], sem_x[ring]
        )

    def out_store(ci, b, ring):
        return pltpu.make_async_copy(
            x_v[ring], out_hbm.at[b, pl.ds(s0 + ci * T, T)], sem_o[ring]
        )

    pos_load(0, 0).start()
    for st in range(3):
        x_load(0, st, st).start()

    def group(g, carry):
        for k in range(8):
            ci = 2 * g + k // 4
            par = (k // 4) % 2
            if k % 4 == 0:
                pos_load(ci, par).wait()
                if k == 0:
                    pos_load(ci + 1, 1 - par).start()
                else:
                    @pl.when(g < NG - 1)
                    def _():
                        pos_load(ci + 1, 1 - par).start()
            x_load(ci, k % 4, k % 4).wait()

            pv = p_v[par]
            xv = x_v[k % 4]

            @plsc.parallel_loop(0, T * D // L, unroll=8)
            def _acc(i):
                r = i // (D // L)
                c = (i % (D // L)) * L
                sl = pl.ds(c, L)
                plsc.addupdate(xv.at[r, sl], pv[r, sl])

            out_store(ci, k % 4, k % 4).start()

            if k >= 1:
                cip, kp = 2 * g + (k - 1) // 4, k - 1
                guard_prev = False
            else:
                cip, kp = 2 * g - 1, 7
                guard_prev = True
            k3 = k + 3
            if k3 < 8:
                ci3, b3 = 2 * g + k3 // 4, k3 % 4
            else:
                ci3, b3 = 2 * g + 2, (k3 - 8) % 4
            ring3 = k3 % 4

            def refill(cip=cip, kp=kp, guard_prev=guard_prev,
                       ci3=ci3, b3=b3, ring3=ring3):
                if guard_prev:
                    @pl.when(g >= 1)
                    def _():
                        out_store(cip, kp % 4, kp % 4).wait()
                else:
                    out_store(cip, kp % 4, kp % 4).wait()
                x_load(ci3, b3, ring3).start()

            if k < 5:
                refill()
            else:
                @pl.when(g < NG - 1)
                def _():
                    refill()
        return carry

    lax.fori_loop(0, NG, group, 0)

    for k in range(4):
        out_store(2 * NG - 1, k, k).wait()


def kernel(x, pos_table):
    return _sc_add(x, pos_table)


# speedup vs baseline: 1.0624x; 1.0058x over previous
import functools

import jax
import jax.numpy as jnp
from jax import lax
from jax.experimental import pallas as pl
from jax.experimental.pallas import tpu as pltpu
from jax.experimental.pallas import tpu_sc as plsc

B, S, D = 4, 4096, 1024
NC, NS, L = 2, 16, 16
NW = NC * NS
SPW = S // NW
T = 16
NCH = SPW // T
NSTEP = NCH * B
NG = NSTEP // 8

_mesh = plsc.VectorSubcoreMesh(
    core_axis_name="c", subcore_axis_name="s", num_cores=NC, num_subcores=NS
)


@functools.partial(
    pl.kernel,
    out_type=jax.ShapeDtypeStruct((B, S, D), jnp.float32),
    mesh=_mesh,
    compiler_params=pltpu.CompilerParams(use_tc_tiling_on_sc=True),
    scratch_types=[
        [pltpu.VMEM((T, D), jnp.float32)] * 2,
        [pltpu.VMEM((T, D), jnp.float32)] * 4,
        [pltpu.SemaphoreType.DMA] * 2,
        [pltpu.SemaphoreType.DMA] * 4,
        [pltpu.SemaphoreType.DMA] * 4,
    ],
)
def _sc_add(x_hbm, pos_hbm, out_hbm, p_v, x_v, sem_p, sem_x, sem_o):
    wid = lax.axis_index("s") * NC + lax.axis_index("c")
    s0 = wid * SPW

    def pos_load(ci, par):
        return pltpu.make_async_copy(
            pos_hbm.at[pl.ds(s0 + ci * T, T)], p_v[par], sem_p[par]
        )

    def x_load(ci, b, ring):
        return pltpu.make_async_copy(
            x_hbm.at[b, pl.ds(s0 + ci * T, T)], x_v[ring], sem_x[ring]
        )

    def out_store(ci, b, ring):
        return pltpu.make_async_copy(
            x_v[ring], out_hbm.at[b, pl.ds(s0 + ci * T, T)], sem_o[ring]
        )

    pos_load(0, 0).start()
    for st in range(3):
        x_load(0, st, st).start()

    def group(g, carry):
        for k in range(8):
            ci = 2 * g + k // 4
            par = (k // 4) % 2
            if k % 4 == 0:
                pos_load(ci, par).wait()
                if k == 0:
                    pos_load(ci + 1, 1 - par).start()
                else:
                    @pl.when(g < NG - 1)
                    def _():
                        pos_load(ci + 1, 1 - par).start()
            x_load(ci, k % 4, k % 4).wait()

            pv = p_v[par]
            xv = x_v[k % 4]

            @plsc.parallel_loop(0, T * D // L, unroll=4)
            def _acc(i):
                r = i // (D // L)
                c = (i % (D // L)) * L
                sl = pl.ds(c, L)
                plsc.addupdate(xv.at[r, sl], pv[r, sl])

            out_store(ci, k % 4, k % 4).start()

            if k >= 1:
                cip, kp = 2 * g + (k - 1) // 4, k - 1
                guard_prev = False
            else:
                cip, kp = 2 * g - 1, 7
                guard_prev = True
            k3 = k + 3
            if k3 < 8:
                ci3, b3 = 2 * g + k3 // 4, k3 % 4
            else:
                ci3, b3 = 2 * g + 2, (k3 - 8) % 4
            ring3 = k3 % 4

            def refill(cip=cip, kp=kp, guard_prev=guard_prev,
                       ci3=ci3, b3=b3, ring3=ring3):
                if guard_prev:
                    @pl.when(g >= 1)
                    def _():
                        out_store(cip, kp % 4, kp % 4).wait()
                else:
                    out_store(cip, kp % 4, kp % 4).wait()
                x_load(ci3, b3, ring3).start()

            if k < 5:
                refill()
            else:
                @pl.when(g < NG - 1)
                def _():
                    refill()
        return carry

    lax.fori_loop(0, NG, group, 0)

    for k in range(4):
        out_store(2 * NG - 1, k, k).wait()


def kernel(x, pos_table):
    return _sc_add(x, pos_table)
